# baseline (speedup 1.0000x reference)
import functools

import jax
import jax.numpy as jnp
from jax import lax
from jax.experimental import pallas as pl
from jax.experimental.pallas import tpu as pltpu
from jax.experimental.pallas import tpu_sc as plsc

N = 10000
E = 320000
D = 128

NC = 2
NS = 16
NW = NC * NS
CSZ = 128
NCH = 80
EPAD = NW * NCH * CSZ
NCH_TOTAL = EPAD // CSZ
NH = 2
HCH = NCH // NH
SLAB = 624
REM = N - NS * SLAB
NJUNK = 16
NACC = N + NJUNK

KB = 8


def _agg_kernel(x_hbm, edges_hbm, zeros_hbm, out_hbm,
                srcv, dstv, buf0, buf1, shared, sem0, sem1):
    cid = lax.axis_index("c")
    sid = lax.axis_index("s")
    wid = cid * NS + sid
    chunk_base = wid * NCH

    pltpu.sync_copy(zeros_hbm, shared.at[pl.ds(sid * SLAB, SLAB)])

    @pl.when(sid == NS - 1)
    def _():
        pltpu.sync_copy(zeros_hbm.at[pl.ds(0, REM + NJUNK)],
                        shared.at[pl.ds(NS * SLAB, REM + NJUNK)])

    plsc.subcore_barrier()

    bufs = (buf0, buf1)
    sems = (sem0, sem1)

    for h in range(NH):
        pltpu.sync_copy(edges_hbm.at[0, pl.ds(chunk_base + h * HCH, HCH)],
                        srcv)
        pltpu.sync_copy(edges_hbm.at[1, pl.ds(chunk_base + h * HCH, HCH)],
                        dstv)

        for b in range(2):
            pltpu.async_copy(x_hbm.at[srcv.at[b]], bufs[b], sems[b])

        @pl.loop(0, HCH, step=KB)
        def _(j0):
            for b in range(KB):
                pltpu.make_async_copy(
                    x_hbm.at[srcv.at[j0 + b]], bufs[b % 2], sems[b % 2]
                ).wait()
                pltpu.sync_copy(bufs[b % 2], shared.at[dstv.at[j0 + b]],
                                add=True)
                nxt = j0 + b + 2

                @pl.when(nxt < HCH)
                def _():
                    pltpu.async_copy(
                        x_hbm.at[srcv.at[nxt]], bufs[b % 2], sems[b % 2])

    plsc.subcore_barrier()

    pltpu.sync_copy(shared.at[pl.ds(sid * SLAB, SLAB)],
                    out_hbm.at[cid, pl.ds(sid * SLAB, SLAB)])

    @pl.when(sid == NS - 1)
    def _():
        pltpu.sync_copy(shared.at[pl.ds(NS * SLAB, REM)],
                        out_hbm.at[cid, pl.ds(NS * SLAB, REM)])


@functools.cache
def _agg():
    return pl.kernel(
        _agg_kernel,
        out_type=jax.ShapeDtypeStruct((NC, N, D), jnp.float32),
        mesh=plsc.VectorSubcoreMesh(core_axis_name="c", subcore_axis_name="s",
                                    num_cores=NC, num_subcores=NS),
        scratch_types=[
            pltpu.VMEM((HCH, CSZ), jnp.int32),
            pltpu.VMEM((HCH, CSZ), jnp.int32),
            pltpu.VMEM((CSZ, D), jnp.float32),
            pltpu.VMEM((CSZ, D), jnp.float32),
            pltpu.VMEM_SHARED((NACC, D), jnp.float32),
            pltpu.SemaphoreType.DMA,
            pltpu.SemaphoreType.DMA,
        ],
    )


BM = 2000


def _dense_kernel(x_ref, p_ref, u_ref, v_ref, o_ref):
    agg = p_ref[0] + p_ref[1]
    dn = (((1,), (1,)), ((), ()))
    acc = lax.dot_general(x_ref[...], u_ref[...], dn,
                          preferred_element_type=jnp.float32)
    acc += lax.dot_general(agg, v_ref[...], dn,
                           preferred_element_type=jnp.float32)
    o_ref[...] = jnp.maximum(acc, 0.0)


def _dense(x, partials, U, V):
    return pl.pallas_call(
        _dense_kernel,
        grid=(N // BM,),
        in_specs=[
            pl.BlockSpec((BM, D), lambda m: (m, 0)),
            pl.BlockSpec((NC, BM, D), lambda m: (0, m, 0)),
            pl.BlockSpec((D, D), lambda m: (0, 0)),
            pl.BlockSpec((D, D), lambda m: (0, 0)),
        ],
        out_specs=pl.BlockSpec((BM, D), lambda m: (m, 0)),
        out_shape=jax.ShapeDtypeStruct((N, D), jnp.float32),
    )(x, partials, U, V)


@jax.jit
def kernel(x, edge_index, U, V):
    npad = EPAD - E
    pad = jnp.stack([
        jnp.zeros((npad,), jnp.int32),
        N + jnp.arange(npad, dtype=jnp.int32) % NJUNK,
    ])
    edges = jnp.concatenate([edge_index, pad], axis=1)
    edges = edges.reshape(2, NCH_TOTAL, CSZ)
    zeros = jnp.zeros((SLAB, D), jnp.float32)
    partials = _agg()(x, edges, zeros)
    return _dense(x, partials, U, V)


# speedup vs baseline: 3.2241x; 3.2241x over previous
import functools

import jax
import jax.numpy as jnp
from jax import lax
from jax.experimental import pallas as pl
from jax.experimental.pallas import tpu as pltpu
from jax.experimental.pallas import tpu_sc as plsc

N = 10000
E = 320000
D = 128

NC = 2
NS = 16
NW = NC * NS
CSZ = 128
NCH = 80
EPAD = NW * NCH * CSZ
NCH_TOTAL = EPAD // CSZ
NH = 2
HCH = NCH // NH
SLAB = 624
REM = N - NS * SLAB
NZROW = 8
NACC = N

KB = 8


def _agg_kernel(x_hbm, edges_hbm, zeros_hbm, out_hbm,
                srcv, dstv, buf0, buf1, shared, sem0, sem1):
    cid = lax.axis_index("c")
    sid = lax.axis_index("s")
    wid = cid * NS + sid
    chunk_base = wid * NCH

    pltpu.sync_copy(zeros_hbm, shared.at[pl.ds(sid * SLAB, SLAB)])

    @pl.when(sid == NS - 1)
    def _():
        pltpu.sync_copy(zeros_hbm.at[pl.ds(0, REM)],
                        shared.at[pl.ds(NS * SLAB, REM)])

    plsc.subcore_barrier()

    bufs = (buf0, buf1)
    sems = (sem0, sem1)

    for h in range(NH):
        pltpu.sync_copy(edges_hbm.at[0, pl.ds(chunk_base + h * HCH, HCH)],
                        srcv)
        pltpu.sync_copy(edges_hbm.at[1, pl.ds(chunk_base + h * HCH, HCH)],
                        dstv)

        for b in range(2):
            pltpu.async_copy(x_hbm.at[srcv.at[b]], bufs[b], sems[b])

        @pl.loop(0, HCH, step=KB)
        def _(j0):
            for b in range(KB):
                pltpu.make_async_copy(
                    x_hbm.at[srcv.at[j0 + b]], bufs[b % 2], sems[b % 2]
                ).wait()
                pltpu.sync_copy(bufs[b % 2], shared.at[dstv.at[j0 + b]],
                                add=True)
                nxt = j0 + b + 2

                @pl.when(nxt < HCH)
                def _():
                    pltpu.async_copy(
                        x_hbm.at[srcv.at[nxt]], bufs[b % 2], sems[b % 2])

    plsc.subcore_barrier()

    pltpu.sync_copy(shared.at[pl.ds(sid * SLAB, SLAB)],
                    out_hbm.at[cid, pl.ds(sid * SLAB, SLAB)])

    @pl.when(sid == NS - 1)
    def _():
        pltpu.sync_copy(shared.at[pl.ds(NS * SLAB, REM)],
                        out_hbm.at[cid, pl.ds(NS * SLAB, REM)])


@functools.cache
def _agg():
    return pl.kernel(
        _agg_kernel,
        out_type=jax.ShapeDtypeStruct((NC, N, D), jnp.float32),
        mesh=plsc.VectorSubcoreMesh(core_axis_name="c", subcore_axis_name="s",
                                    num_cores=NC, num_subcores=NS),
        scratch_types=[
            pltpu.VMEM((HCH, CSZ), jnp.int32),
            pltpu.VMEM((HCH, CSZ), jnp.int32),
            pltpu.VMEM((CSZ, D), jnp.float32),
            pltpu.VMEM((CSZ, D), jnp.float32),
            pltpu.VMEM_SHARED((N, D), jnp.float32),
            pltpu.SemaphoreType.DMA,
            pltpu.SemaphoreType.DMA,
        ],
    )


BM = 2000


def _dense_kernel(x_ref, p_ref, u_ref, v_ref, o_ref):
    agg = p_ref[0] + p_ref[1]
    dn = (((1,), (1,)), ((), ()))
    acc = lax.dot_general(x_ref[...], u_ref[...], dn,
                          preferred_element_type=jnp.float32)
    acc += lax.dot_general(agg, v_ref[...], dn,
                           preferred_element_type=jnp.float32)
    o_ref[...] = jnp.maximum(acc, 0.0)


def _dense(x, partials, U, V):
    return pl.pallas_call(
        _dense_kernel,
        grid=(N // BM,),
        in_specs=[
            pl.BlockSpec((BM, D), lambda m: (m, 0)),
            pl.BlockSpec((NC, BM, D), lambda m: (0, m, 0)),
            pl.BlockSpec((D, D), lambda m: (0, 0)),
            pl.BlockSpec((D, D), lambda m: (0, 0)),
        ],
        out_specs=pl.BlockSpec((BM, D), lambda m: (m, 0)),
        out_shape=jax.ShapeDtypeStruct((N, D), jnp.float32),
    )(x, partials, U, V)


@jax.jit
def kernel(x, edge_index, U, V):
    npad = EPAD - E
    iota = jnp.arange(npad, dtype=jnp.int32)
    pad = jnp.stack([N + iota % NZROW, iota % N])
    edges = jnp.concatenate([edge_index, pad], axis=1)
    edges = edges.reshape(2, NCH_TOTAL, CSZ)
    zeros = jnp.zeros((SLAB, D), jnp.float32)
    x_ext = jnp.concatenate([x, jnp.zeros((NZROW, D), jnp.float32)])
    partials = _agg()(x_ext, edges, zeros)
    return _dense(x, partials, U, V)


# speedup vs baseline: 3.5371x; 1.0971x over previous
import functools

import jax
import jax.numpy as jnp
from jax import lax
from jax.experimental import pallas as pl
from jax.experimental.pallas import tpu as pltpu
from jax.experimental.pallas import tpu_sc as plsc

N = 10000
E = 320000
D = 128

NC = 2
NS = 16
NW = NC * NS
EDGES_PER_TILE = E // NW
CSZ = 125
NCH = EDGES_PER_TILE // CSZ
NH = 2
HCH = NCH // NH
NCH_TOTAL = E // CSZ
SLAB = 624
REM = N - NS * SLAB


KB = 8


def _agg_kernel(x_hbm, src_hbm, dst_hbm, zeros_hbm, out_hbm,
                srcv, dstv, buf0, buf1, shared, sem0, sem1):
    cid = lax.axis_index("c")
    sid = lax.axis_index("s")
    wid = cid * NS + sid
    chunk_base = wid * NCH

    pltpu.sync_copy(zeros_hbm, shared.at[pl.ds(sid * SLAB, SLAB)])

    @pl.when(sid == NS - 1)
    def _():
        pltpu.sync_copy(zeros_hbm.at[pl.ds(0, REM)],
                        shared.at[pl.ds(NS * SLAB, REM)])

    plsc.subcore_barrier()

    bufs = (buf0, buf1)
    sems = (sem0, sem1)

    for h in range(NH):
        pltpu.sync_copy(src_hbm.at[pl.ds(chunk_base + h * HCH, HCH)], srcv)
        pltpu.sync_copy(dst_hbm.at[pl.ds(chunk_base + h * HCH, HCH)], dstv)

        for b in range(2):
            pltpu.async_copy(x_hbm.at[srcv.at[b]], bufs[b], sems[b])

        @pl.loop(0, HCH, step=KB)
        def _(j0):
            for b in range(KB):
                pltpu.make_async_copy(
                    x_hbm.at[srcv.at[j0 + b]], bufs[b % 2], sems[b % 2]
                ).wait()
                pltpu.sync_copy(bufs[b % 2], shared.at[dstv.at[j0 + b]],
                                add=True)
                nxt = j0 + b + 2

                @pl.when(nxt < HCH)
                def _():
                    pltpu.async_copy(
                        x_hbm.at[srcv.at[nxt]], bufs[b % 2], sems[b % 2])

    plsc.subcore_barrier()

    pltpu.sync_copy(shared.at[pl.ds(sid * SLAB, SLAB)],
                    out_hbm.at[cid, pl.ds(sid * SLAB, SLAB)])

    @pl.when(sid == NS - 1)
    def _():
        pltpu.sync_copy(shared.at[pl.ds(NS * SLAB, REM)],
                        out_hbm.at[cid, pl.ds(NS * SLAB, REM)])


@functools.cache
def _agg():
    return pl.kernel(
        _agg_kernel,
        out_type=jax.ShapeDtypeStruct((NC, N, D), jnp.float32),
        mesh=plsc.VectorSubcoreMesh(core_axis_name="c", subcore_axis_name="s",
                                    num_cores=NC, num_subcores=NS),
        scratch_types=[
            pltpu.VMEM((HCH, CSZ), jnp.int32),
            pltpu.VMEM((HCH, CSZ), jnp.int32),
            pltpu.VMEM((CSZ, D), jnp.float32),
            pltpu.VMEM((CSZ, D), jnp.float32),
            pltpu.VMEM_SHARED((N, D), jnp.float32),
            pltpu.SemaphoreType.DMA,
            pltpu.SemaphoreType.DMA,
        ],
    )


BM = 1000

DN = (((1,), (1,)), ((), ()))


def _xu_kernel(x_ref, u_ref, o_ref):
    o_ref[...] = lax.dot_general(x_ref[...], u_ref[...], DN,
                                 preferred_element_type=jnp.float32)


def _xu(x, U):
    return pl.pallas_call(
        _xu_kernel,
        grid=(N // BM,),
        in_specs=[
            pl.BlockSpec((BM, D), lambda m: (m, 0)),
            pl.BlockSpec((D, D), lambda m: (0, 0)),
        ],
        out_specs=pl.BlockSpec((BM, D), lambda m: (m, 0)),
        out_shape=jax.ShapeDtypeStruct((N, D), jnp.float32),
    )(x, U)


def _combine_kernel(d_ref, p_ref, v_ref, o_ref):
    agg = p_ref[0] + p_ref[1]
    acc = d_ref[...] + lax.dot_general(agg, v_ref[...], DN,
                                       preferred_element_type=jnp.float32)
    o_ref[...] = jnp.maximum(acc, 0.0)


def _combine(dst_term, partials, V):
    return pl.pallas_call(
        _combine_kernel,
        grid=(N // BM,),
        in_specs=[
            pl.BlockSpec((BM, D), lambda m: (m, 0)),
            pl.BlockSpec((NC, BM, D), lambda m: (0, m, 0)),
            pl.BlockSpec((D, D), lambda m: (0, 0)),
        ],
        out_specs=pl.BlockSpec((BM, D), lambda m: (m, 0)),
        out_shape=jax.ShapeDtypeStruct((N, D), jnp.float32),
    )(dst_term, partials, V)


@jax.jit
def kernel(x, edge_index, U, V):
    src2 = edge_index[0].reshape(NCH_TOTAL, CSZ)
    dst2 = edge_index[1].reshape(NCH_TOTAL, CSZ)
    zeros = jnp.zeros((SLAB, D), jnp.float32)
    dst_term = _xu(x, U)
    partials = _agg()(x, src2, dst2, zeros)
    return _combine(dst_term, partials, V)


# speedup vs baseline: 3.6628x; 1.0355x over previous
import functools

import jax
import jax.numpy as jnp
from jax import lax
from jax.experimental import pallas as pl
from jax.experimental.pallas import tpu as pltpu
from jax.experimental.pallas import tpu_sc as plsc

N = 10000
E = 320000
D = 128

NC = 2
NS = 16
NW = NC * NS
EDGES_PER_TILE = E // NW
CSZ = 125
NCH = EDGES_PER_TILE // CSZ
NH = 2
HCH = NCH // NH
NCH_TOTAL = E // CSZ
SLAB = 624
REM = N - NS * SLAB


KB = 8


def _agg_kernel(x_hbm, src_hbm, dst_hbm, zeros_hbm, out_hbm,
                srcv, dstv, buf0, buf1, shared, sem0, sem1):
    cid = lax.axis_index("c")
    sid = lax.axis_index("s")
    wid = cid * NS + sid
    chunk_base = wid * NCH

    idx0 = pltpu.async_copy(src_hbm.at[pl.ds(chunk_base, HCH)], srcv, sem0)
    idx1 = pltpu.async_copy(dst_hbm.at[pl.ds(chunk_base, HCH)], dstv, sem1)

    pltpu.sync_copy(zeros_hbm, shared.at[pl.ds(sid * SLAB, SLAB)])

    @pl.when(sid == NS - 1)
    def _():
        pltpu.sync_copy(zeros_hbm.at[pl.ds(0, REM)],
                        shared.at[pl.ds(NS * SLAB, REM)])

    plsc.subcore_barrier()

    bufs = (buf0, buf1)
    sems = (sem0, sem1)

    for h in range(NH):
        if h > 0:
            pltpu.sync_copy(src_hbm.at[pl.ds(chunk_base + h * HCH, HCH)],
                            srcv)
            pltpu.sync_copy(dst_hbm.at[pl.ds(chunk_base + h * HCH, HCH)],
                            dstv)
        else:
            idx0.wait()
            idx1.wait()

        for b in range(2):
            pltpu.async_copy(x_hbm.at[srcv.at[b]], bufs[b], sems[b])

        @pl.loop(0, HCH, step=KB)
        def _(j0):
            for b in range(KB):
                pltpu.make_async_copy(
                    x_hbm.at[srcv.at[j0 + b]], bufs[b % 2], sems[b % 2]
                ).wait()
                pltpu.sync_copy(bufs[b % 2], shared.at[dstv.at[j0 + b]],
                                add=True)
                nxt = j0 + b + 2

                @pl.when(nxt < HCH)
                def _():
                    pltpu.async_copy(
                        x_hbm.at[srcv.at[nxt]], bufs[b % 2], sems[b % 2])

    plsc.subcore_barrier()

    pltpu.sync_copy(shared.at[pl.ds(sid * SLAB, SLAB)],
                    out_hbm.at[cid, pl.ds(sid * SLAB, SLAB)])

    @pl.when(sid == NS - 1)
    def _():
        pltpu.sync_copy(shared.at[pl.ds(NS * SLAB, REM)],
                        out_hbm.at[cid, pl.ds(NS * SLAB, REM)])


@functools.cache
def _agg():
    return pl.kernel(
        _agg_kernel,
        out_type=jax.ShapeDtypeStruct((NC, N, D), jnp.float32),
        mesh=plsc.VectorSubcoreMesh(core_axis_name="c", subcore_axis_name="s",
                                    num_cores=NC, num_subcores=NS),
        scratch_types=[
            pltpu.VMEM((HCH, CSZ), jnp.int32),
            pltpu.VMEM((HCH, CSZ), jnp.int32),
            pltpu.VMEM((CSZ, D), jnp.float32),
            pltpu.VMEM((CSZ, D), jnp.float32),
            pltpu.VMEM_SHARED((N, D), jnp.float32),
            pltpu.SemaphoreType.DMA,
            pltpu.SemaphoreType.DMA,
        ],
    )


BM = 2000


def _dense_kernel(x_ref, p_ref, u_ref, v_ref, o_ref):
    agg = p_ref[0] + p_ref[1]
    dn = (((1,), (1,)), ((), ()))
    acc = lax.dot_general(x_ref[...], u_ref[...], dn,
                          preferred_element_type=jnp.float32)
    acc += lax.dot_general(agg, v_ref[...], dn,
                           preferred_element_type=jnp.float32)
    o_ref[...] = jnp.maximum(acc, 0.0)


def _dense(x, partials, U, V):
    return pl.pallas_call(
        _dense_kernel,
        grid=(N // BM,),
        in_specs=[
            pl.BlockSpec((BM, D), lambda m: (m, 0)),
            pl.BlockSpec((NC, BM, D), lambda m: (0, m, 0)),
            pl.BlockSpec((D, D), lambda m: (0, 0)),
            pl.BlockSpec((D, D), lambda m: (0, 0)),
        ],
        out_specs=pl.BlockSpec((BM, D), lambda m: (m, 0)),
        out_shape=jax.ShapeDtypeStruct((N, D), jnp.float32),
    )(x, partials, U, V)


@jax.jit
def kernel(x, edge_index, U, V):
    src2 = edge_index[0].reshape(NCH_TOTAL, CSZ)
    dst2 = edge_index[1].reshape(NCH_TOTAL, CSZ)
    zeros = jnp.zeros((SLAB, D), jnp.float32)
    partials = _agg()(x, src2, dst2, zeros)
    return _dense(x, partials, U, V)


# speedup vs baseline: 3.6662x; 1.0009x over previous
import functools

import jax
import jax.numpy as jnp
from jax import lax
from jax.experimental import pallas as pl
from jax.experimental.pallas import tpu as pltpu
from jax.experimental.pallas import tpu_sc as plsc

N = 10000
E = 320000
D = 128

NC = 2
NS = 16
NW = NC * NS
EDGES_PER_TILE = E // NW
CSZ = 125
NCH = EDGES_PER_TILE // CSZ
NH = 2
HCH = NCH // NH
NCH_TOTAL = E // CSZ
SLAB = 624
REM = N - NS * SLAB


KB = 10


def _agg_kernel(x_hbm, src_hbm, dst_hbm, zeros_hbm, out_hbm,
                srcv, dstv, buf0, buf1, shared, sem0, sem1):
    cid = lax.axis_index("c")
    sid = lax.axis_index("s")
    wid = cid * NS + sid
    chunk_base = wid * NCH

    idx0 = pltpu.async_copy(src_hbm.at[pl.ds(chunk_base, HCH)], srcv, sem0)
    idx1 = pltpu.async_copy(dst_hbm.at[pl.ds(chunk_base, HCH)], dstv, sem1)

    pltpu.sync_copy(zeros_hbm, shared.at[pl.ds(sid * SLAB, SLAB)])

    @pl.when(sid == NS - 1)
    def _():
        pltpu.sync_copy(zeros_hbm.at[pl.ds(0, REM)],
                        shared.at[pl.ds(NS * SLAB, REM)])

    plsc.subcore_barrier()

    bufs = (buf0, buf1)
    sems = (sem0, sem1)

    for h in range(NH):
        if h > 0:
            pltpu.sync_copy(src_hbm.at[pl.ds(chunk_base + h * HCH, HCH)],
                            srcv)
            pltpu.sync_copy(dst_hbm.at[pl.ds(chunk_base + h * HCH, HCH)],
                            dstv)
        else:
            idx0.wait()
            idx1.wait()

        for b in range(2):
            pltpu.async_copy(x_hbm.at[srcv.at[b]], bufs[b], sems[b])

        @pl.loop(0, HCH, step=KB)
        def _(j0):
            for b in range(KB):
                pltpu.make_async_copy(
                    x_hbm.at[srcv.at[j0 + b]], bufs[b % 2], sems[b % 2]
                ).wait()
                pltpu.sync_copy(bufs[b % 2], shared.at[dstv.at[j0 + b]],
                                add=True)
                nxt = j0 + b + 2

                @pl.when(nxt < HCH)
                def _():
                    pltpu.async_copy(
                        x_hbm.at[srcv.at[nxt]], bufs[b % 2], sems[b % 2])

    plsc.subcore_barrier()

    pltpu.sync_copy(shared.at[pl.ds(sid * SLAB, SLAB)],
                    out_hbm.at[cid, pl.ds(sid * SLAB, SLAB)])

    @pl.when(sid == NS - 1)
    def _():
        pltpu.sync_copy(shared.at[pl.ds(NS * SLAB, REM)],
                        out_hbm.at[cid, pl.ds(NS * SLAB, REM)])


@functools.cache
def _agg():
    return pl.kernel(
        _agg_kernel,
        out_type=jax.ShapeDtypeStruct((NC, N, D), jnp.float32),
        mesh=plsc.VectorSubcoreMesh(core_axis_name="c", subcore_axis_name="s",
                                    num_cores=NC, num_subcores=NS),
        scratch_types=[
            pltpu.VMEM((HCH, CSZ), jnp.int32),
            pltpu.VMEM((HCH, CSZ), jnp.int32),
            pltpu.VMEM((CSZ, D), jnp.float32),
            pltpu.VMEM((CSZ, D), jnp.float32),
            pltpu.VMEM_SHARED((N, D), jnp.float32),
            pltpu.SemaphoreType.DMA,
            pltpu.SemaphoreType.DMA,
        ],
    )


BM = 2000


def _dense_kernel(x_ref, p_ref, u_ref, v_ref, o_ref):
    agg = p_ref[0] + p_ref[1]
    dn = (((1,), (1,)), ((), ()))
    acc = lax.dot_general(x_ref[...], u_ref[...], dn,
                          preferred_element_type=jnp.float32)
    acc += lax.dot_general(agg, v_ref[...], dn,
                           preferred_element_type=jnp.float32)
    o_ref[...] = jnp.maximum(acc, 0.0)


def _dense(x, partials, U, V):
    return pl.pallas_call(
        _dense_kernel,
        grid=(N // BM,),
        in_specs=[
            pl.BlockSpec((BM, D), lambda m: (m, 0)),
            pl.BlockSpec((NC, BM, D), lambda m: (0, m, 0)),
            pl.BlockSpec((D, D), lambda m: (0, 0)),
            pl.BlockSpec((D, D), lambda m: (0, 0)),
        ],
        out_specs=pl.BlockSpec((BM, D), lambda m: (m, 0)),
        out_shape=jax.ShapeDtypeStruct((N, D), jnp.float32),
    )(x, partials, U, V)


@jax.jit
def kernel(x, edge_index, U, V):
    src2 = edge_index[0].reshape(NCH_TOTAL, CSZ)
    dst2 = edge_index[1].reshape(NCH_TOTAL, CSZ)
    zeros = jnp.zeros((SLAB, D), jnp.float32)
    partials = _agg()(x, src2, dst2, zeros)
    return _dense(x, partials, U, V)
